# baseline (speedup 1.0000x reference)
import numpy as np
import jax
import jax.numpy as jnp
from jax import lax
from jax.experimental import pallas as pl
from jax.experimental.pallas import tpu as pltpu

_WIN = 6


def _dct_colsum():
    n = np.arange(_WIN)
    D = np.cos(np.pi * (n[None, :] + 0.5) * n[:, None] / _WIN)
    scale = np.full(_WIN, np.sqrt(2.0 / _WIN))
    scale[0] = np.sqrt(1.0 / _WIN)
    D = D * scale[:, None]
    return tuple(float(v) for v in D.sum(axis=0))


_TSUM = _dct_colsum()


def kernel(x, W1, b1, W2, b2):
    B, L, C = x.shape
    S = L // _WIN
    H = W1.shape[2]
    CB = 256
    n_cb = (C + CB - 1) // CB
    Bb = 8
    nb = B // Bb
    G = L // 24

    w1p = jnp.repeat(W1.transpose(2, 1, 0), _WIN, axis=1)
    w2p = jnp.repeat(W2.transpose(1, 2, 0), _WIN, axis=1)
    b1t = b1.T
    b2p = jnp.repeat(b2.T, _WIN, axis=0)

    def body(x_ref, w1p_ref, b1_ref, w2p_ref, b2p_ref, o_ref, btp_scr):
        bi = pl.program_id(1)

        @pl.when(bi == 0)
        def _():
            betap = b2p_ref[...]
            for h in range(H):
                betap = betap + b1_ref[h:h + 1, :] * w2p_ref[h]
            rows = lax.broadcasted_iota(jnp.int32, (24, CB), 0)
            ph = rows % _WIN
            tp24 = jnp.full((24, CB), _TSUM[0], jnp.float32)
            for n in range(1, _WIN):
                tp24 = jnp.where(ph == n, _TSUM[n], tp24)
            btp_scr[...] = betap * pltpu.repeat(tp24, G, axis=0)

        HL = L // 2

        def tree_sum24(parts):
            while len(parts) > 1:
                nxt = [parts[i] + parts[i + 1] for i in range(0, len(parts) - 1, 2)]
                if len(parts) % 2:
                    nxt.append(parts[-1])
                parts = nxt
            return parts[0]

        def per_b(b, carry):
            xb = x_ref[b]
            u24x = tree_sum24([xb[24 * g:24 * g + 24] for g in range(G)])
            mean_b = jnp.sum(u24x, axis=0, keepdims=True) * (1.0 / L)
            us = [None] * H
            for q in range(2):
                xc_q = x_ref[b, q * HL:(q + 1) * HL, :] - mean_b
                for h in range(H):
                    pr = xc_q * w1p_ref[h, q * HL:(q + 1) * HL]
                    t = tree_sum24([pr[24 * g:24 * g + 24] for g in range(G // 2)])
                    us[h] = t if us[h] is None else us[h] + t
            hps = []
            for h in range(H):
                u = us[h]
                u = u + jnp.concatenate([u[12:], u[:12]], axis=0)
                hp = u + jnp.concatenate([u[6:], u[:6]], axis=0)
                hps.append(hp)
            for q in range(2):
                out_q = btp_scr[q * HL:(q + 1) * HL] + mean_b
                for h in range(H):
                    out_q = out_q + (pltpu.repeat(hps[h], G // 2, axis=0)
                                     * w2p_ref[h, q * HL:(q + 1) * HL])
                o_ref[b, q * HL:(q + 1) * HL, :] = out_q
            return carry

        jax.lax.fori_loop(0, Bb, per_b, 0)

    out = pl.pallas_call(
        body,
        out_shape=jax.ShapeDtypeStruct((B, L, C), jnp.float32),
        grid=(n_cb, nb),
        in_specs=[
            pl.BlockSpec((Bb, L, CB), lambda c, bi: (bi, 0, c)),
            pl.BlockSpec((H, L, CB), lambda c, bi: (0, 0, c)),
            pl.BlockSpec((H, CB), lambda c, bi: (0, c)),
            pl.BlockSpec((H, L, CB), lambda c, bi: (0, 0, c)),
            pl.BlockSpec((L, CB), lambda c, bi: (0, c)),
        ],
        out_specs=pl.BlockSpec((Bb, L, CB), lambda c, bi: (bi, 0, c)),
        scratch_shapes=[
            pltpu.VMEM((L, CB), jnp.float32),
        ],
        compiler_params=pltpu.CompilerParams(
            dimension_semantics=("arbitrary", "arbitrary"),
            vmem_limit_bytes=52 * 1024 * 1024,
        ),
        name="esn_ltf_fused",
    )(x, w1p, b1t, w2p, b2p)

    return out


# speedup vs baseline: 1.1452x; 1.1452x over previous
import numpy as np
import jax
import jax.numpy as jnp
from jax import lax
from jax.experimental import pallas as pl
from jax.experimental.pallas import tpu as pltpu

_WIN = 6


def _dct_colsum():
    n = np.arange(_WIN)
    D = np.cos(np.pi * (n[None, :] + 0.5) * n[:, None] / _WIN)
    scale = np.full(_WIN, np.sqrt(2.0 / _WIN))
    scale[0] = np.sqrt(1.0 / _WIN)
    D = D * scale[:, None]
    return tuple(float(v) for v in D.sum(axis=0))


_TSUM = _dct_colsum()


def kernel(x, W1, b1, W2, b2):
    B, L, C = x.shape
    S = L // _WIN
    H = W1.shape[2]
    CB = 128
    n_cb = (C + CB - 1) // CB
    Bb = 16
    nb = B // Bb
    G = L // 24

    w1p = jnp.repeat(W1.transpose(2, 1, 0), _WIN, axis=1)
    w2p = jnp.repeat(W2.transpose(1, 2, 0), _WIN, axis=1)
    b1t = b1.T
    b2p = jnp.repeat(b2.T, _WIN, axis=0)

    def body(x_ref, w1p_ref, b1_ref, w2p_ref, b2p_ref, o_ref, btp_scr):
        bi = pl.program_id(1)

        @pl.when(bi == 0)
        def _():
            betap = b2p_ref[...]
            for h in range(H):
                betap = betap + b1_ref[h:h + 1, :] * w2p_ref[h]
            rows = lax.broadcasted_iota(jnp.int32, (24, CB), 0)
            ph = rows % _WIN
            tp24 = jnp.full((24, CB), _TSUM[0], jnp.float32)
            for n in range(1, _WIN):
                tp24 = jnp.where(ph == n, _TSUM[n], tp24)
            btp_scr[...] = betap * pltpu.repeat(tp24, G, axis=0)

        HL = L // 2

        def tree_sum24(parts):
            while len(parts) > 1:
                nxt = [parts[i] + parts[i + 1] for i in range(0, len(parts) - 1, 2)]
                if len(parts) % 2:
                    nxt.append(parts[-1])
                parts = nxt
            return parts[0]

        def per_b(b, carry):
            xb = x_ref[b]
            u24x = tree_sum24([xb[24 * g:24 * g + 24] for g in range(G)])
            mean_b = jnp.sum(u24x, axis=0, keepdims=True) * (1.0 / L)
            us = [None] * H
            for q in range(2):
                xc_q = x_ref[b, q * HL:(q + 1) * HL, :] - mean_b
                for h in range(H):
                    pr = xc_q * w1p_ref[h, q * HL:(q + 1) * HL]
                    t = tree_sum24([pr[24 * g:24 * g + 24] for g in range(G // 2)])
                    us[h] = t if us[h] is None else us[h] + t
            hps = []
            for h in range(H):
                u = us[h]
                u = u + jnp.concatenate([u[12:], u[:12]], axis=0)
                hp = u + jnp.concatenate([u[6:], u[:6]], axis=0)
                hps.append(hp)
            for q in range(2):
                out_q = btp_scr[q * HL:(q + 1) * HL] + mean_b
                for h in range(H):
                    out_q = out_q + (pltpu.repeat(hps[h], G // 2, axis=0)
                                     * w2p_ref[h, q * HL:(q + 1) * HL])
                o_ref[b, q * HL:(q + 1) * HL, :] = out_q
            return carry

        jax.lax.fori_loop(0, Bb, per_b, 0)

    out = pl.pallas_call(
        body,
        out_shape=jax.ShapeDtypeStruct((B, L, C), jnp.float32),
        grid=(n_cb, nb),
        in_specs=[
            pl.BlockSpec((Bb, L, CB), lambda c, bi: (bi, 0, c)),
            pl.BlockSpec((H, L, CB), lambda c, bi: (0, 0, c)),
            pl.BlockSpec((H, CB), lambda c, bi: (0, c)),
            pl.BlockSpec((H, L, CB), lambda c, bi: (0, 0, c)),
            pl.BlockSpec((L, CB), lambda c, bi: (0, c)),
        ],
        out_specs=pl.BlockSpec((Bb, L, CB), lambda c, bi: (bi, 0, c)),
        scratch_shapes=[
            pltpu.VMEM((L, CB), jnp.float32),
        ],
        compiler_params=pltpu.CompilerParams(
            dimension_semantics=("arbitrary", "arbitrary"),
        ),
        name="esn_ltf_fused",
    )(x, w1p, b1t, w2p, b2p)

    return out
